# baseline (speedup 1.0000x reference)
import functools

import jax
import jax.numpy as jnp
from jax import lax
from jax.experimental import pallas as pl
from jax.experimental.pallas import tpu as pltpu
from jax.experimental.pallas import tpu_sc as plsc

N = 50000
B = 128
H = 256
NC, NS = 2, 16
NW = NC * NS
C = 48
FULL = N // C
TAIL = N - FULL * C
ITERS = (FULL + NW - 1) // NW
OUTER = (ITERS + 1) // 2

_mesh = plsc.VectorSubcoreMesh(core_axis_name="c", subcore_axis_name="s")


@functools.partial(
    pl.kernel,
    out_type=jax.ShapeDtypeStruct((NW, 2, B, H), jnp.float32),
    mesh=_mesh,
    scratch_types=[
        pltpu.VMEM((C,), jnp.int32),
        pltpu.VMEM((C,), jnp.int32),
        pltpu.VMEM((2 * C, H), jnp.float32),
        pltpu.VMEM((2 * C, H), jnp.float32),
        pltpu.VMEM((2 * B, H), jnp.float32),
        pltpu.SemaphoreType.DMA,
        pltpu.SemaphoreType.DMA,
        pltpu.SemaphoreType.DMA,
        pltpu.SemaphoreType.DMA,
    ],
)
def _segment_sums_sc(node_hbm, edge_hbm, idx_hbm, out_hbm,
                     idx0_v, idx1_v, r0_v, r1_v, acc_v,
                     si0, si1, sr0, sr1):
    cid = lax.axis_index("c")
    sid = lax.axis_index("s")
    wid = sid * NC + cid

    idx_bufs = (idx0_v, idx1_v)
    r_bufs = (r0_v, r1_v)
    sems = ((si0, sr0), (si1, sr1))

    zero = jnp.zeros((16,), jnp.float32)
    def zero_body(r, carry):
        for j in range(H // 16):
            acc_v[r, pl.ds(j * 16, 16)] = zero
        return carry
    lax.fori_loop(0, 2 * B, zero_body, 0)

    def issue(i, b):
        k = wid + i * NW
        @pl.when(k < FULL)
        def _():
            base = k * C
            pltpu.async_copy(idx_hbm.at[pl.ds(base, C)], idx_bufs[b], sems[b][0])
            pltpu.async_copy(node_hbm.at[pl.ds(base, C)],
                             r_bufs[b].at[pl.ds(0, C)], sems[b][1])
            pltpu.async_copy(edge_hbm.at[pl.ds(base, C)],
                             r_bufs[b].at[pl.ds(C, C)], sems[b][1])

    def wait(i, b):
        k = wid + i * NW
        @pl.when(k < FULL)
        def _():
            pltpu.make_async_copy(idx_hbm.at[pl.ds(0, C)], idx_bufs[b], sems[b][0]).wait()
            pltpu.make_async_copy(node_hbm.at[pl.ds(0, C)],
                                  r_bufs[b].at[pl.ds(0, C)], sems[b][1]).wait()
            pltpu.make_async_copy(edge_hbm.at[pl.ds(0, C)],
                                  r_bufs[b].at[pl.ds(C, C)], sems[b][1]).wait()

    def accumulate(idx_ref, rows_ref, nrows):
        GN = nrows // 16

        @plsc.parallel_loop(0, 2 * GN, unroll=2)
        def _(g):
            is_edge = g >= GN
            gi = jnp.where(is_edge, g - GN, g)
            off = jnp.where(is_edge, B, 0)
            segs = idx_ref[pl.ds(gi * 16, 16)]
            for l in range(16):
                seg = segs[l] + off
                r = g * 16 + l
                vals = [rows_ref[r, pl.ds(j * 16, 16)] for j in range(H // 16)]
                for j in range(H // 16):
                    plsc.addupdate(acc_v.at[seg, pl.ds(j * 16, 16)], vals[j])

    issue(0, 0)
    issue(1, 1)

    def outer_body(t, carry):
        for b in range(2):
            i = 2 * t + b
            k = wid + i * NW
            wait(i, b)
            @pl.when(k < FULL)
            def _():
                accumulate(idx_bufs[b], r_bufs[b], C)
            issue(i + 2, b)
        return carry

    lax.fori_loop(0, OUTER, outer_body, 0)

    @pl.when(wid == NW - 1)
    def _():
        base = FULL * C
        pltpu.sync_copy(idx_hbm.at[pl.ds(base, TAIL)], idx0_v.at[pl.ds(0, TAIL)])
        pltpu.sync_copy(node_hbm.at[pl.ds(base, TAIL)], r0_v.at[pl.ds(0, TAIL)])
        pltpu.sync_copy(edge_hbm.at[pl.ds(base, TAIL)], r0_v.at[pl.ds(TAIL, TAIL)])
        accumulate(idx0_v, r0_v, TAIL)

    pltpu.sync_copy(acc_v.at[pl.ds(0, B)], out_hbm.at[wid, 0])
    pltpu.sync_copy(acc_v.at[pl.ds(B, B)], out_hbm.at[wid, 1])


def _softplus(x):
    return jnp.maximum(x, 0.0) + jnp.log1p(jnp.exp(-jnp.abs(x)))


def _mlp_tc(g_ref, p_ref, wc_ref, bc_ref, w1_ref, b1_ref, w2_ref, b2_ref,
            w3_ref, b3_ref, out_ref):
    na = jnp.sum(p_ref[:, 0], axis=0)
    ea = jnp.sum(p_ref[:, 1], axis=0)
    wc = wc_ref[...]
    dot = functools.partial(jnp.dot, preferred_element_type=jnp.float32,
                            precision=lax.Precision.HIGHEST)
    comb = (dot(g_ref[...], wc[0:H])
            + dot(na, wc[H:2 * H])
            + dot(ea, wc[2 * H:3 * H])
            + bc_ref[...])
    h = _softplus(dot(comb, w1_ref[...]) + b1_ref[...])
    h = _softplus(dot(h, w2_ref[...]) + b2_ref[...])
    out_ref[...] = dot(h, w3_ref[...]) + b3_ref[...]


def kernel(global_feat, node_features, edge_features, batch_idx,
           W_c, b_c, W1, b1, W2, b2, W3, b3):
    idx = batch_idx.astype(jnp.int32)
    partials = _segment_sums_sc(node_features, edge_features, idx)
    out = pl.pallas_call(
        _mlp_tc,
        out_shape=jax.ShapeDtypeStruct((B, H), jnp.float32),
    )(global_feat, partials,
      W_c, b_c.reshape(1, H), W1, b1.reshape(1, H),
      W2, b2.reshape(1, H), W3, b3.reshape(1, H))
    return out


# speedup vs baseline: 1.2706x; 1.2706x over previous
import functools

import jax
import jax.numpy as jnp
from jax import lax
from jax.experimental import pallas as pl
from jax.experimental.pallas import tpu as pltpu
from jax.experimental.pallas import tpu_sc as plsc

N = 50000
B = 128
H = 256
NC, NS = 2, 16
NW = NC * NS
C = 48
FULL = N // C
TAIL = N - FULL * C
ITERS = (FULL + NW - 1) // NW
OUTER = (ITERS + 1) // 2

_mesh = plsc.VectorSubcoreMesh(core_axis_name="c", subcore_axis_name="s")


@functools.partial(
    pl.kernel,
    out_type=jax.ShapeDtypeStruct((NW, 2, B, H), jnp.float32),
    mesh=_mesh,
    scratch_types=[
        pltpu.VMEM((C,), jnp.int32),
        pltpu.VMEM((C,), jnp.int32),
        pltpu.VMEM((2 * C, H), jnp.float32),
        pltpu.VMEM((2 * C, H), jnp.float32),
        pltpu.VMEM((2 * B, H), jnp.float32),
        pltpu.SMEM((2 * C,), jnp.int32),
        pltpu.SemaphoreType.DMA,
        pltpu.SemaphoreType.DMA,
        pltpu.SemaphoreType.DMA,
        pltpu.SemaphoreType.DMA,
    ],
)
def _segment_sums_sc(node_hbm, edge_hbm, idx_hbm, out_hbm,
                     idx0_v, idx1_v, r0_v, r1_v, acc_v, segs_s,
                     si0, si1, sr0, sr1):
    cid = lax.axis_index("c")
    sid = lax.axis_index("s")
    wid = sid * NC + cid

    idx_bufs = (idx0_v, idx1_v)
    r_bufs = (r0_v, r1_v)
    sems = ((si0, sr0), (si1, sr1))

    zero = jnp.zeros((16,), jnp.float32)
    def zero_body(r, carry):
        for j in range(H // 16):
            acc_v[r, pl.ds(j * 16, 16)] = zero
        return carry
    lax.fori_loop(0, 2 * B, zero_body, 0)

    def issue(i, b):
        k = wid + i * NW
        @pl.when(k < FULL)
        def _():
            base = k * C
            pltpu.async_copy(idx_hbm.at[pl.ds(base, C)], idx_bufs[b], sems[b][0])
            pltpu.async_copy(node_hbm.at[pl.ds(base, C)],
                             r_bufs[b].at[pl.ds(0, C)], sems[b][1])
            pltpu.async_copy(edge_hbm.at[pl.ds(base, C)],
                             r_bufs[b].at[pl.ds(C, C)], sems[b][1])

    def wait(i, b):
        k = wid + i * NW
        @pl.when(k < FULL)
        def _():
            pltpu.make_async_copy(idx_hbm.at[pl.ds(0, C)], idx_bufs[b], sems[b][0]).wait()
            pltpu.make_async_copy(node_hbm.at[pl.ds(0, C)],
                                  r_bufs[b].at[pl.ds(0, C)], sems[b][1]).wait()
            pltpu.make_async_copy(edge_hbm.at[pl.ds(0, C)],
                                  r_bufs[b].at[pl.ds(C, C)], sems[b][1]).wait()

    def accumulate(idx_ref, rows_ref, nrows):
        GN = nrows // 16

        def seg_prepass(g, carry):
            segs = idx_ref[pl.ds(g * 16, 16)]
            for l in range(16):
                segs_s[g * 16 + l] = segs[l]
                segs_s[GN * 16 + g * 16 + l] = segs[l] + B
            return carry
        lax.fori_loop(0, GN, seg_prepass, 0)

        @plsc.parallel_loop(0, 2 * nrows, unroll=4)
        def _(r):
            seg = segs_s[r]
            vals = [rows_ref[r, pl.ds(j * 16, 16)] for j in range(H // 16)]
            for j in range(H // 16):
                plsc.addupdate(acc_v.at[seg, pl.ds(j * 16, 16)], vals[j])

    issue(0, 0)
    issue(1, 1)

    def outer_body(t, carry):
        for b in range(2):
            i = 2 * t + b
            k = wid + i * NW
            wait(i, b)
            @pl.when(k < FULL)
            def _():
                accumulate(idx_bufs[b], r_bufs[b], C)
            issue(i + 2, b)
        return carry

    lax.fori_loop(0, OUTER, outer_body, 0)

    @pl.when(wid == NW - 1)
    def _():
        base = FULL * C
        pltpu.sync_copy(idx_hbm.at[pl.ds(base, TAIL)], idx0_v.at[pl.ds(0, TAIL)])
        pltpu.sync_copy(node_hbm.at[pl.ds(base, TAIL)], r0_v.at[pl.ds(0, TAIL)])
        pltpu.sync_copy(edge_hbm.at[pl.ds(base, TAIL)], r0_v.at[pl.ds(TAIL, TAIL)])
        accumulate(idx0_v, r0_v, TAIL)

    pltpu.sync_copy(acc_v.at[pl.ds(0, B)], out_hbm.at[wid, 0])
    pltpu.sync_copy(acc_v.at[pl.ds(B, B)], out_hbm.at[wid, 1])


def _softplus(x):
    return jnp.maximum(x, 0.0) + jnp.log1p(jnp.exp(-jnp.abs(x)))


def _mlp_tc(g_ref, p_ref, wc_ref, bc_ref, w1_ref, b1_ref, w2_ref, b2_ref,
            w3_ref, b3_ref, out_ref):
    na = jnp.sum(p_ref[:, 0], axis=0)
    ea = jnp.sum(p_ref[:, 1], axis=0)
    wc = wc_ref[...]
    dot = functools.partial(jnp.dot, preferred_element_type=jnp.float32,
                            precision=lax.Precision.HIGHEST)
    comb = (dot(g_ref[...], wc[0:H])
            + dot(na, wc[H:2 * H])
            + dot(ea, wc[2 * H:3 * H])
            + bc_ref[...])
    h = _softplus(dot(comb, w1_ref[...]) + b1_ref[...])
    h = _softplus(dot(h, w2_ref[...]) + b2_ref[...])
    out_ref[...] = dot(h, w3_ref[...]) + b3_ref[...]


def kernel(global_feat, node_features, edge_features, batch_idx,
           W_c, b_c, W1, b1, W2, b2, W3, b3):
    idx = batch_idx.astype(jnp.int32)
    partials = _segment_sums_sc(node_features, edge_features, idx)
    out = pl.pallas_call(
        _mlp_tc,
        out_shape=jax.ShapeDtypeStruct((B, H), jnp.float32),
    )(global_feat, partials,
      W_c, b_c.reshape(1, H), W1, b1.reshape(1, H),
      W2, b2.reshape(1, H), W3, b3.reshape(1, H))
    return out


# speedup vs baseline: 1.4640x; 1.1522x over previous
import functools

import jax
import jax.numpy as jnp
from jax import lax
from jax.experimental import pallas as pl
from jax.experimental.pallas import tpu as pltpu
from jax.experimental.pallas import tpu_sc as plsc

N = 50000
B = 128
H = 256
NC, NS = 2, 16
NW = NC * NS
C = 48
FULL = N // C
TAIL = N - FULL * C
ITERS = (FULL + NW - 1) // NW
NFULLW = FULL - (ITERS - 1) * NW
OUTER = (ITERS + 1) // 2
IDXLEN = ITERS * C

_mesh = plsc.VectorSubcoreMesh(core_axis_name="c", subcore_axis_name="s")


@functools.partial(
    pl.kernel,
    out_type=jax.ShapeDtypeStruct((NW, 2, B, H), jnp.float32),
    mesh=_mesh,
    scratch_types=[
        pltpu.VMEM((IDXLEN + TAIL,), jnp.int32),
        pltpu.VMEM((2 * C, H), jnp.float32),
        pltpu.VMEM((2 * C, H), jnp.float32),
        pltpu.VMEM((2 * B, H), jnp.float32),
        pltpu.SemaphoreType.DMA,
        pltpu.SemaphoreType.DMA,
        pltpu.SemaphoreType.DMA,
    ],
)
def _segment_sums_sc(node_hbm, edge_hbm, idx_hbm, out_hbm,
                     idx_v, r0_v, r1_v, acc_v,
                     si, sr0, sr1):
    cid = lax.axis_index("c")
    sid = lax.axis_index("s")
    wid = sid * NC + cid

    r_bufs = (r0_v, r1_v)
    rsems = (sr0, sr1)
    start = ITERS * jnp.minimum(wid, NFULLW) \
        + (ITERS - 1) * jnp.maximum(wid - NFULLW, 0)
    cnt = jnp.where(wid < NFULLW, ITERS, ITERS - 1)
    ibase = jnp.minimum(start * C, N - (IDXLEN + TAIL))
    delta = start * C - ibase
    pltpu.async_copy(idx_hbm.at[pl.ds(ibase, IDXLEN + TAIL)], idx_v, si)

    zero = jnp.zeros((16,), jnp.float32)
    def zero_body(r, carry):
        for j in range(H // 16):
            acc_v[r, pl.ds(j * 16, 16)] = zero
        return carry
    lax.fori_loop(0, 2 * B, zero_body, 0)

    def issue(i, b):
        @pl.when(i < cnt)
        def _():
            base = (start + i) * C
            pltpu.async_copy(node_hbm.at[pl.ds(base, C)],
                             r_bufs[b].at[pl.ds(0, C)], rsems[b])
            pltpu.async_copy(edge_hbm.at[pl.ds(base, C)],
                             r_bufs[b].at[pl.ds(C, C)], rsems[b])

    def wait(i, b):
        @pl.when(i < cnt)
        def _():
            pltpu.make_async_copy(node_hbm.at[pl.ds(0, C)],
                                  r_bufs[b].at[pl.ds(0, C)], rsems[b]).wait()
            pltpu.make_async_copy(edge_hbm.at[pl.ds(0, C)],
                                  r_bufs[b].at[pl.ds(C, C)], rsems[b]).wait()

    def accumulate(idx_off, rows_ref, nrows):
        GN = nrows // 16
        NJ = H // 16
        zerov = jnp.zeros((16,), jnp.float32)

        def grp_body(g, carry):
            is_edge = g >= GN
            gi = jnp.where(is_edge, g - GN, g)
            off = jnp.where(is_edge, B, 0)
            base_r = g * 16
            segs = idx_v[pl.ds(idx_off + gi * 16, 16)]
            s0 = segs[0] + off
            s15 = segs[15] + off

            @pl.when(s0 == s15)
            def _():
                for jb in range(NJ // 4):
                    accs = [zerov] * 4
                    for l in range(16):
                        vals = [rows_ref[base_r + l, pl.ds((jb * 4 + jj) * 16, 16)]
                                for jj in range(4)]
                        accs = [accs[jj] + vals[jj] for jj in range(4)]
                    for jj in range(4):
                        plsc.addupdate(acc_v.at[s0, pl.ds((jb * 4 + jj) * 16, 16)],
                                       accs[jj])

            @pl.when(s0 != s15)
            def _():
                segs2 = idx_v[pl.ds(idx_off + gi * 16, 16)]
                for l in range(16):
                    seg = segs2[l] + off
                    vals = [rows_ref[base_r + l, pl.ds(j * 16, 16)]
                            for j in range(NJ)]
                    for j in range(NJ):
                        plsc.addupdate(acc_v.at[seg, pl.ds(j * 16, 16)], vals[j])
            return carry

        lax.fori_loop(0, 2 * GN, grp_body, 0)

    pltpu.make_async_copy(idx_hbm.at[pl.ds(0, IDXLEN + TAIL)], idx_v, si).wait()
    issue(0, 0)
    issue(1, 1)

    def outer_body(t, carry):
        for b in range(2):
            i = 2 * t + b
            wait(i, b)
            @pl.when(i < cnt)
            def _():
                accumulate(delta + i * C, r_bufs[b], C)
            issue(i + 2, b)
        return carry

    lax.fori_loop(0, OUTER, outer_body, 0)

    @pl.when(wid == NW - 1)
    def _():
        base = FULL * C
        pltpu.sync_copy(node_hbm.at[pl.ds(base, TAIL)], r0_v.at[pl.ds(0, TAIL)])
        pltpu.sync_copy(edge_hbm.at[pl.ds(base, TAIL)], r0_v.at[pl.ds(TAIL, TAIL)])
        accumulate(delta + cnt * C, r0_v, TAIL)

    pltpu.sync_copy(acc_v.at[pl.ds(0, B)], out_hbm.at[wid, 0])
    pltpu.sync_copy(acc_v.at[pl.ds(B, B)], out_hbm.at[wid, 1])


def _softplus(x):
    return jnp.maximum(x, 0.0) + jnp.log1p(jnp.exp(-jnp.abs(x)))


def _mlp_tc(g_ref, p_ref, wc_ref, bc_ref, w1_ref, b1_ref, w2_ref, b2_ref,
            w3_ref, b3_ref, out_ref):
    na = jnp.sum(p_ref[:, 0], axis=0)
    ea = jnp.sum(p_ref[:, 1], axis=0)
    wc = wc_ref[...]
    dot = functools.partial(jnp.dot, preferred_element_type=jnp.float32,
                            precision=lax.Precision.HIGHEST)
    comb = (dot(g_ref[...], wc[0:H])
            + dot(na, wc[H:2 * H])
            + dot(ea, wc[2 * H:3 * H])
            + bc_ref[...])
    h = _softplus(dot(comb, w1_ref[...]) + b1_ref[...])
    h = _softplus(dot(h, w2_ref[...]) + b2_ref[...])
    out_ref[...] = dot(h, w3_ref[...]) + b3_ref[...]


def kernel(global_feat, node_features, edge_features, batch_idx,
           W_c, b_c, W1, b1, W2, b2, W3, b3):
    idx = batch_idx.astype(jnp.int32)
    partials = _segment_sums_sc(node_features, edge_features, idx)
    out = pl.pallas_call(
        _mlp_tc,
        out_shape=jax.ShapeDtypeStruct((B, H), jnp.float32),
    )(global_feat, partials,
      W_c, b_c.reshape(1, H), W1, b1.reshape(1, H),
      W2, b2.reshape(1, H), W3, b3.reshape(1, H))
    return out


# speedup vs baseline: 1.4696x; 1.0038x over previous
import functools

import jax
import jax.numpy as jnp
from jax import lax
from jax.experimental import pallas as pl
from jax.experimental.pallas import tpu as pltpu
from jax.experimental.pallas import tpu_sc as plsc

N = 50000
B = 128
H = 256
NC, NS = 2, 16
NW = NC * NS
C = 48
FULL = N // C
TAIL = N - FULL * C
ITERS = (FULL + NW - 1) // NW
OUTER = (ITERS + 1) // 2

_mesh = plsc.VectorSubcoreMesh(core_axis_name="c", subcore_axis_name="s")


@functools.partial(
    pl.kernel,
    out_type=jax.ShapeDtypeStruct((NW, 2, B, H), jnp.float32),
    mesh=_mesh,
    scratch_types=[
        pltpu.VMEM((C,), jnp.int32),
        pltpu.VMEM((C,), jnp.int32),
        pltpu.VMEM((2 * C, H), jnp.float32),
        pltpu.VMEM((2 * C, H), jnp.float32),
        pltpu.VMEM((2 * B, H), jnp.float32),
        pltpu.SemaphoreType.DMA,
        pltpu.SemaphoreType.DMA,
        pltpu.SemaphoreType.DMA,
        pltpu.SemaphoreType.DMA,
    ],
)
def _segment_sums_sc(node_hbm, edge_hbm, idx_hbm, out_hbm,
                     idx0_v, idx1_v, r0_v, r1_v, acc_v,
                     si0, si1, sr0, sr1):
    cid = lax.axis_index("c")
    sid = lax.axis_index("s")
    wid = sid * NC + cid

    idx_bufs = (idx0_v, idx1_v)
    r_bufs = (r0_v, r1_v)
    sems = ((si0, sr0), (si1, sr1))

    zero = jnp.zeros((16,), jnp.float32)
    def zero_body(r, carry):
        for j in range(H // 16):
            acc_v[r, pl.ds(j * 16, 16)] = zero
        return carry
    lax.fori_loop(0, 2 * B, zero_body, 0)

    def issue(i, b):
        k = wid + i * NW
        @pl.when(k < FULL)
        def _():
            base = k * C
            pltpu.async_copy(idx_hbm.at[pl.ds(base, C)], idx_bufs[b], sems[b][0])
            pltpu.async_copy(node_hbm.at[pl.ds(base, C)],
                             r_bufs[b].at[pl.ds(0, C)], sems[b][1])
            pltpu.async_copy(edge_hbm.at[pl.ds(base, C)],
                             r_bufs[b].at[pl.ds(C, C)], sems[b][1])

    def wait(i, b):
        k = wid + i * NW
        @pl.when(k < FULL)
        def _():
            pltpu.make_async_copy(idx_hbm.at[pl.ds(0, C)], idx_bufs[b], sems[b][0]).wait()
            pltpu.make_async_copy(node_hbm.at[pl.ds(0, C)],
                                  r_bufs[b].at[pl.ds(0, C)], sems[b][1]).wait()
            pltpu.make_async_copy(edge_hbm.at[pl.ds(0, C)],
                                  r_bufs[b].at[pl.ds(C, C)], sems[b][1]).wait()

    def accumulate(idx_ref, rows_ref, nrows):
        GN = nrows // 16
        NJ = H // 16
        zerov = jnp.zeros((16,), jnp.float32)

        def grp_body(g, carry):
            is_edge = g >= GN
            gi = jnp.where(is_edge, g - GN, g)
            off = jnp.where(is_edge, B, 0)
            base_r = g * 16
            segs = idx_ref[pl.ds(gi * 16, 16)]
            s0 = segs[0] + off
            s15 = segs[15] + off

            @pl.when(s0 == s15)
            def _():
                for jb in range(NJ // 4):
                    accs = [zerov] * 4
                    for l in range(16):
                        vals = [rows_ref[base_r + l, pl.ds((jb * 4 + jj) * 16, 16)]
                                for jj in range(4)]
                        accs = [accs[jj] + vals[jj] for jj in range(4)]
                    for jj in range(4):
                        plsc.addupdate(acc_v.at[s0, pl.ds((jb * 4 + jj) * 16, 16)],
                                       accs[jj])

            @pl.when(s0 != s15)
            def _():
                segs2 = idx_ref[pl.ds(gi * 16, 16)]
                for l in range(16):
                    seg = segs2[l] + off
                    vals = [rows_ref[base_r + l, pl.ds(j * 16, 16)]
                            for j in range(NJ)]
                    for j in range(NJ):
                        plsc.addupdate(acc_v.at[seg, pl.ds(j * 16, 16)], vals[j])
            return carry

        lax.fori_loop(0, 2 * GN, grp_body, 0)

    issue(0, 0)
    issue(1, 1)

    def outer_body(t, carry):
        for b in range(2):
            i = 2 * t + b
            k = wid + i * NW
            wait(i, b)
            @pl.when(k < FULL)
            def _():
                accumulate(idx_bufs[b], r_bufs[b], C)
            issue(i + 2, b)
        return carry

    lax.fori_loop(0, OUTER, outer_body, 0)

    @pl.when(wid == NW - 1)
    def _():
        base = FULL * C
        pltpu.sync_copy(idx_hbm.at[pl.ds(base, TAIL)], idx0_v.at[pl.ds(0, TAIL)])
        pltpu.sync_copy(node_hbm.at[pl.ds(base, TAIL)], r0_v.at[pl.ds(0, TAIL)])
        pltpu.sync_copy(edge_hbm.at[pl.ds(base, TAIL)], r0_v.at[pl.ds(TAIL, TAIL)])
        accumulate(idx0_v, r0_v, TAIL)

    pltpu.sync_copy(acc_v.at[pl.ds(0, B)], out_hbm.at[wid, 0])
    pltpu.sync_copy(acc_v.at[pl.ds(B, B)], out_hbm.at[wid, 1])


def _softplus(x):
    return jnp.maximum(x, 0.0) + jnp.log1p(jnp.exp(-jnp.abs(x)))


def _mlp_tc(g_ref, p_ref, wc_ref, bc_ref, w1_ref, b1_ref, w2_ref, b2_ref,
            w3_ref, b3_ref, out_ref):
    na = jnp.sum(p_ref[:, 0], axis=0)
    ea = jnp.sum(p_ref[:, 1], axis=0)
    wc = wc_ref[...]
    dot = functools.partial(jnp.dot, preferred_element_type=jnp.float32,
                            precision=lax.Precision.HIGHEST)
    comb = (dot(g_ref[...], wc[0:H])
            + dot(na, wc[H:2 * H])
            + dot(ea, wc[2 * H:3 * H])
            + bc_ref[...])
    h = _softplus(dot(comb, w1_ref[...]) + b1_ref[...])
    h = _softplus(dot(h, w2_ref[...]) + b2_ref[...])
    out_ref[...] = dot(h, w3_ref[...]) + b3_ref[...]


def kernel(global_feat, node_features, edge_features, batch_idx,
           W_c, b_c, W1, b1, W2, b2, W3, b3):
    idx = batch_idx.astype(jnp.int32)
    partials = _segment_sums_sc(node_features, edge_features, idx)
    out = pl.pallas_call(
        _mlp_tc,
        out_shape=jax.ShapeDtypeStruct((B, H), jnp.float32),
    )(global_feat, partials,
      W_c, b_c.reshape(1, H), W1, b1.reshape(1, H),
      W2, b2.reshape(1, H), W3, b3.reshape(1, H))
    return out
